# baseline (speedup 1.0000x reference)
import functools

import jax
import jax.numpy as jnp
from jax import lax
from jax.experimental import pallas as pl
from jax.experimental.pallas import tpu as pltpu
from jax.experimental.pallas import tpu_sc as plsc

V_SIZE = 1000000
EMB_SIZE = 64
BATCH = 16384

NC = 2
NS = 16
LANES = 16
K_PER_C = EMB_SIZE // NC
B_PER_S = BATCH // NS
N_LISTS = B_PER_S // 128
H0 = 499968
H1 = V_SIZE - H0


def _sc_partials(focus, context, emb_t):
  mesh = plsc.VectorSubcoreMesh(core_axis_name="c", subcore_axis_name="s")

  @functools.partial(
      pl.kernel,
      out_type=jax.ShapeDtypeStruct((NC * NS, LANES), jnp.float32),
      mesh=mesh,
      scratch_types=[
          pltpu.VMEM_SHARED((H0,), jnp.float32),
          pltpu.VMEM_SHARED((H1,), jnp.float32),
          pltpu.VMEM((B_PER_S,), jnp.int32),
          pltpu.VMEM((B_PER_S,), jnp.int32),
          pltpu.VMEM((B_PER_S,), jnp.int32),
          pltpu.VMEM((B_PER_S,), jnp.int32),
          pltpu.VMEM((B_PER_S,), jnp.float32),
          pltpu.VMEM((B_PER_S,), jnp.float32),
          pltpu.VMEM((B_PER_S,), jnp.float32),
          pltpu.VMEM((B_PER_S,), jnp.float32),
          pltpu.VMEM((B_PER_S,), jnp.float32),
          pltpu.VMEM((B_PER_S,), jnp.float32),
          pltpu.VMEM((LANES,), jnp.float32),
          pltpu.SemaphoreType.DMA,
          pltpu.SemaphoreType.DMA,
          pltpu.SemaphoreType.DMA,
      ],
  )
  def body(focus_hbm, ctx_hbm, emb_hbm, out_hbm,
           row0, row1, if0, if1, ic0, ic1, mf, mc,
           gfa, gfb, gca, gcb, partial_v, semA, semB, semG):
    c = lax.axis_index("c")
    s = lax.axis_index("s")
    wid = s * NC + c
    seg = s * B_PER_S
    k0 = c * K_PER_C

    pltpu.sync_copy(focus_hbm.at[pl.ds(seg, B_PER_S)], if0)
    pltpu.sync_copy(ctx_hbm.at[pl.ds(seg, B_PER_S)], ic0)

    def split_body(g, _):
      sl = pl.ds(g * LANES, LANES)
      for i0, i1, msk in ((if0, if1, mf), (ic0, ic1, mc)):
        raw = i0[sl]
        i1[sl] = jnp.clip(raw - H0, 0, H1 - 1)
        msk[sl] = jnp.where(raw < H0, 1.0, 0.0).astype(jnp.float32)
        i0[sl] = jnp.minimum(raw, H0 - 1)
      return 0

    lax.fori_loop(0, B_PER_S // LANES, split_body, 0)

    n_sub = EMB_SIZE // LANES

    def start_half(kk, half):
      if half == 0:
        pltpu.async_copy(emb_hbm.at[k0 + kk, pl.ds(0, H0)], row0, semA)
      else:
        pltpu.async_copy(emb_hbm.at[k0 + kk, pl.ds(H0, H1)], row1, semB)

    def drain_half(half):
      if half == 0:
        pltpu.make_async_copy(emb_hbm.at[0, pl.ds(0, H0)], row0, semA).wait()
      else:
        pltpu.make_async_copy(emb_hbm.at[0, pl.ds(H0, H1)], row1, semB).wait()

    def gather_half(row, idx_fr, idx_cr, gf, gc):
      for idx_ref, g_ref in ((idx_fr, gf), (idx_cr, gc)):
        for q in range(N_LISTS):
          sl = pl.ds(q * 128, 128)
          pltpu.async_copy(row.at[idx_ref.at[sl]], g_ref.at[sl], semG)
      dummy = emb_hbm.at[0, pl.ds(0, B_PER_S)]
      pltpu.make_async_copy(dummy, gf, semG).wait()
      pltpu.make_async_copy(dummy, gc, semG).wait()

    def combine_acc(accs):
      accs = list(accs)
      for m in range(B_PER_S // LANES):
        sl = pl.ds(m * LANES, LANES)
        fb = gfb[sl]
        cb = gcb[sl]
        f = fb + mf[sl] * (gfa[sl] - fb)
        cc = cb + mc[sl] * (gca[sl] - cb)
        accs[m % n_sub] = accs[m % n_sub] + f * cc
      return tuple(accs)

    @pl.when(s == 0)
    def _():
      start_half(0, 0)
      start_half(0, 1)

    def feat_body(kk, accs):
      @pl.when(s == 0)
      def _():
        drain_half(0)
      plsc.subcore_barrier()
      gather_half(row0, if0, ic0, gfa, gca)
      plsc.subcore_barrier()
      @pl.when(jnp.logical_and(s == 0, kk + 1 < K_PER_C))
      def _():
        start_half(kk + 1, 0)
      @pl.when(s == 0)
      def _():
        drain_half(1)
      plsc.subcore_barrier()
      gather_half(row1, if1, ic1, gfb, gcb)
      accs = combine_acc(accs)
      plsc.subcore_barrier()
      @pl.when(jnp.logical_and(s == 0, kk + 1 < K_PER_C))
      def _():
        start_half(kk + 1, 1)
      return accs

    zero = jnp.zeros((LANES,), jnp.float32)
    accs = lax.fori_loop(0, K_PER_C, feat_body, (zero,) * n_sub)

    total = accs[0]
    for j in range(1, n_sub):
      total = total + accs[j]
    partial_v[...] = total
    pltpu.sync_copy(partial_v, out_hbm.at[wid])

  return body(focus, context, emb_t)


def _finalize(partials):
  def tc_body(p_ref, o_ref):
    s = jnp.sum(p_ref[...])
    ls = jnp.minimum(s, 0.0) - jnp.log(1.0 + jnp.exp(-jnp.abs(s)))
    o_ref[...] = jnp.reshape(ls, (1, 1))

  return pl.pallas_call(
      tc_body,
      out_shape=jax.ShapeDtypeStruct((1, 1), jnp.float32),
  )(partials)


@jax.jit
def kernel(focus, context, emb):
  emb_t = jnp.transpose(emb)
  partials = _sc_partials(focus, context, emb_t)
  return _finalize(partials)


# speedup vs baseline: 1.7578x; 1.7578x over previous
import functools

import jax
import jax.numpy as jnp
from jax import lax
from jax.experimental import pallas as pl
from jax.experimental.pallas import tpu as pltpu
from jax.experimental.pallas import tpu_sc as plsc

V_SIZE = 1000000
EMB_SIZE = 64
BATCH = 16384

NC = 2
NS = 16
LANES = 16
NW = NC * NS
B_PER_W = BATCH // NW
CHUNK = 16
N_CHUNKS = B_PER_W // CHUNK


def _sc_partials(focus, context, emb):
  mesh = plsc.VectorSubcoreMesh(core_axis_name="c", subcore_axis_name="s")

  @functools.partial(
      pl.kernel,
      out_type=jax.ShapeDtypeStruct((NW, LANES), jnp.float32),
      mesh=mesh,
      scratch_types=[
          pltpu.VMEM((B_PER_W,), jnp.int32),
          pltpu.VMEM((B_PER_W,), jnp.int32),
          pltpu.VMEM((2, CHUNK, EMB_SIZE), jnp.float32),
          pltpu.VMEM((2, CHUNK, EMB_SIZE), jnp.float32),
          pltpu.VMEM((LANES,), jnp.float32),
          pltpu.SemaphoreType.DMA,
          pltpu.SemaphoreType.DMA,
      ],
  )
  def body(focus_hbm, ctx_hbm, emb_hbm, out_hbm,
           idx_f, idx_c, bf, bc, partial_v, sem0, sem1):
    wid = lax.axis_index("s") * NC + lax.axis_index("c")
    base = wid * B_PER_W

    pltpu.sync_copy(focus_hbm.at[pl.ds(base, B_PER_W)], idx_f)
    pltpu.sync_copy(ctx_hbm.at[pl.ds(base, B_PER_W)], idx_c)

    n_sub = EMB_SIZE // LANES
    sems = (sem0, sem1)

    def start_chunk(c, slot):
      rv_f = idx_f[pl.ds(c * CHUNK, CHUNK)]
      rv_c = idx_c[pl.ds(c * CHUNK, CHUNK)]
      for l in range(CHUNK):
        pltpu.async_copy(emb_hbm.at[rv_f[l]], bf.at[slot, l], sems[slot])
        pltpu.async_copy(emb_hbm.at[rv_c[l]], bc.at[slot, l], sems[slot])

    def wait_chunk(slot):
      dummy = emb_hbm.at[0]
      for l in range(CHUNK):
        pltpu.make_async_copy(dummy, bf.at[slot, l], sems[slot]).wait()
        pltpu.make_async_copy(dummy, bc.at[slot, l], sems[slot]).wait()

    def compute_chunk(slot, accs):
      accs = list(accs)
      for l in range(CHUNK):
        for j in range(n_sub):
          accs[j] = accs[j] + (bf[slot, l, pl.ds(j * LANES, LANES)]
                               * bc[slot, l, pl.ds(j * LANES, LANES)])
      return tuple(accs)

    start_chunk(0, 0)
    start_chunk(1, 1)

    def pair_body(g, accs):
      c0 = g * 2
      wait_chunk(0)
      @pl.when(c0 + 2 < N_CHUNKS)
      def _():
        start_chunk(c0 + 2, 0)
      accs = compute_chunk(0, accs)
      wait_chunk(1)
      @pl.when(c0 + 3 < N_CHUNKS)
      def _():
        start_chunk(c0 + 3, 1)
      accs = compute_chunk(1, accs)
      return accs

    zero = jnp.zeros((LANES,), jnp.float32)
    accs = lax.fori_loop(0, N_CHUNKS // 2, pair_body, (zero,) * n_sub)

    total = accs[0]
    for j in range(1, n_sub):
      total = total + accs[j]
    partial_v[...] = total
    pltpu.sync_copy(partial_v, out_hbm.at[wid])

  return body(focus, context, emb)


def _finalize(partials):
  def tc_body(p_ref, o_ref):
    s = jnp.sum(p_ref[...])
    ls = jnp.minimum(s, 0.0) - jnp.log(1.0 + jnp.exp(-jnp.abs(s)))
    o_ref[...] = jnp.reshape(ls, (1, 1))

  return pl.pallas_call(
      tc_body,
      out_shape=jax.ShapeDtypeStruct((1, 1), jnp.float32),
  )(partials)


@jax.jit
def kernel(focus, context, emb):
  partials = _sc_partials(focus, context, emb)
  return _finalize(partials)
